# Initial kernel scaffold; baseline (speedup 1.0000x reference)
import jax
import jax.numpy as jnp
from jax.experimental import pallas as pl


def kernel(x, wt, bias, lora_A, bt):
    raise NotImplementedError("write your pallas kernel here")


# speedup vs baseline: 1.8982x; 1.8982x over previous
import jax
import jax.numpy as jnp
from jax.experimental import pallas as pl
from jax.experimental.pallas import tpu as pltpu


def _lora_matmul_kernel(x_ref, w_ref, xa_ref, bt_ref, bias_ref, o_ref):
    acc = jnp.dot(x_ref[...], w_ref[...], preferred_element_type=jnp.float32)
    lora = jnp.dot(xa_ref[...], bt_ref[...], preferred_element_type=jnp.float32)
    o_ref[...] = acc + lora + bias_ref[...]


def kernel(x, wt, bias, lora_A, bt):
    *lead, in_f = x.shape
    out_f = wt.shape[1]
    rank = bt.shape[0]

    x2 = x.reshape(-1, in_f)
    m = x2.shape[0]

    xb = x2.astype(jnp.bfloat16)
    wb = wt.astype(jnp.bfloat16)
    atb = lora_A.T.astype(jnp.bfloat16)
    btb = bt.astype(jnp.bfloat16)
    xa = jnp.dot(xb, atb, preferred_element_type=jnp.float32).astype(jnp.bfloat16)

    tm, tn = 1024, 1024
    grid = (m // tm, out_f // tn)

    flops = 2 * m * in_f * out_f + 2 * m * in_f * rank + 2 * m * rank * out_f
    bytes_accessed = 2 * (m * in_f + (out_f // tn) * 0
                          + in_f * out_f * (m // tm)
                          + m * rank + rank * out_f) + 4 * (out_f + m * out_f)

    out = pl.pallas_call(
        _lora_matmul_kernel,
        out_shape=jax.ShapeDtypeStruct((m, out_f), x.dtype),
        grid=grid,
        in_specs=[
            pl.BlockSpec((tm, in_f), lambda i, j: (i, 0)),
            pl.BlockSpec((in_f, tn), lambda i, j: (0, j)),
            pl.BlockSpec((tm, rank), lambda i, j: (i, 0)),
            pl.BlockSpec((rank, tn), lambda i, j: (0, j)),
            pl.BlockSpec((1, tn), lambda i, j: (0, j)),
        ],
        out_specs=pl.BlockSpec((tm, tn), lambda i, j: (i, j)),
        compiler_params=pltpu.CompilerParams(
            dimension_semantics=("parallel", "arbitrary"),
            vmem_limit_bytes=60 * 1024 * 1024,
        ),
        cost_estimate=pl.CostEstimate(
            flops=flops, transcendentals=0, bytes_accessed=bytes_accessed),
    )(xb, wb, xa, btb, bias)

    return out.reshape(*lead, out_f)


# speedup vs baseline: 2.1222x; 1.1180x over previous
import jax
import jax.numpy as jnp
from jax.experimental import pallas as pl
from jax.experimental.pallas import tpu as pltpu


def _fused_lora_kernel(x_ref, w_ref, at_ref, bt_ref, bias_ref, o_ref, xa_ref):
    j = pl.program_id(1)
    xb = x_ref[...].astype(jnp.bfloat16)

    @pl.when(j == 0)
    def _():
        xa_ref[...] = jnp.dot(
            xb, at_ref[...], preferred_element_type=jnp.float32
        ).astype(jnp.bfloat16)

    acc = jnp.dot(xb, w_ref[...].astype(jnp.bfloat16),
                  preferred_element_type=jnp.float32)
    lora = jnp.dot(xa_ref[...], bt_ref[...],
                   preferred_element_type=jnp.float32)
    o_ref[...] = acc + lora + bias_ref[...]


def kernel(x, wt, bias, lora_A, bt):
    *lead, in_f = x.shape
    out_f = wt.shape[1]
    rank = bt.shape[0]

    x2 = x.reshape(-1, in_f)
    m = x2.shape[0]

    atb = lora_A.T.astype(jnp.bfloat16)
    btb = bt.astype(jnp.bfloat16)

    tm, tn = 1024, 512
    grid = (m // tm, out_f // tn)

    flops = 2 * m * in_f * out_f + 2 * m * in_f * rank + 2 * m * rank * out_f
    bytes_accessed = 4 * (m * in_f + in_f * out_f * (m // tm)
                          + out_f + m * out_f) + 2 * (in_f + out_f) * rank

    out = pl.pallas_call(
        _fused_lora_kernel,
        out_shape=jax.ShapeDtypeStruct((m, out_f), x.dtype),
        grid=grid,
        in_specs=[
            pl.BlockSpec((tm, in_f), lambda i, j: (i, 0)),
            pl.BlockSpec((in_f, tn), lambda i, j: (0, j)),
            pl.BlockSpec((in_f, rank), lambda i, j: (0, 0)),
            pl.BlockSpec((rank, tn), lambda i, j: (0, j)),
            pl.BlockSpec((1, tn), lambda i, j: (0, j)),
        ],
        out_specs=pl.BlockSpec((tm, tn), lambda i, j: (i, j)),
        scratch_shapes=[pltpu.VMEM((tm, rank), jnp.bfloat16)],
        compiler_params=pltpu.CompilerParams(
            dimension_semantics=("parallel", "arbitrary"),
            vmem_limit_bytes=62 * 1024 * 1024,
        ),
        cost_estimate=pl.CostEstimate(
            flops=flops, transcendentals=0, bytes_accessed=bytes_accessed),
    )(x2, wt, atb, btb, bias)

    return out.reshape(*lead, out_f)
